# Initial kernel scaffold; baseline (speedup 1.0000x reference)
import jax
import jax.numpy as jnp
from jax.experimental import pallas as pl


def kernel(hidden, times_table, interval_table, connection_table, W1, b1, W2, b2, source, destination, times, intervals, connection_types):
    raise NotImplementedError("write your pallas kernel here")


# speedup vs baseline: 1.3355x; 1.3355x over previous
import functools

import jax
import jax.numpy as jnp
from jax import lax
from jax.experimental import pallas as pl
from jax.experimental.pallas import tpu as pltpu
from jax.experimental.pallas import tpu_sc as plsc

NC = 2
NS = 16
NW = NC * NS
L = 16


def _premul_nodes(hidden, W1s, W1d):
    N, D = hidden.shape
    R = 1000
    assert N % R == 0

    def body(h_ref, ws_ref, wd_ref, u_ref, v_ref):
        h = h_ref[...]
        u_ref[...] = jnp.dot(h, ws_ref[...], preferred_element_type=jnp.float32)
        v_ref[...] = jnp.dot(h, wd_ref[...], preferred_element_type=jnp.float32)

    return pl.pallas_call(
        body,
        grid=(N // R,),
        in_specs=[
            pl.BlockSpec((R, D), lambda i: (i, 0)),
            pl.BlockSpec((D, D), lambda i: (0, 0)),
            pl.BlockSpec((D, D), lambda i: (0, 0)),
        ],
        out_specs=[pl.BlockSpec((R, D), lambda i: (i, 0))] * 2,
        out_shape=[jax.ShapeDtypeStruct((N, D), jnp.float32)] * 2,
    )(hidden, W1s, W1d)


def _premul_side(times_table, interval_table, connection_table, W1t, W1i, W1c, b1):
    NT, TH = times_table.shape
    NI = interval_table.shape[0]
    NCN = connection_table.shape[0]
    D = W1t.shape[1]

    def body(tt, it, ct, wt, wi, wc, b1r, tp, ip, cp):
        b = b1r[...]
        tp[...] = jnp.dot(tt[...], wt[...], preferred_element_type=jnp.float32) + b
        ip[...] = jnp.dot(it[...], wi[...], preferred_element_type=jnp.float32)
        cp[...] = jnp.dot(ct[...], wc[...], preferred_element_type=jnp.float32)

    return pl.pallas_call(
        body,
        out_shape=[
            jax.ShapeDtypeStruct((NT, D), jnp.float32),
            jax.ShapeDtypeStruct((NI, D), jnp.float32),
            jax.ShapeDtypeStruct((NCN, D), jnp.float32),
        ],
    )(times_table, interval_table, connection_table, W1t, W1i, W1c,
      b1.reshape(1, D))


def _make_sc_score(B, D, NT, NI, NCN, E):
    per_w = B // NW
    chunks = per_w // E
    groups = E // L
    mesh = plsc.VectorSubcoreMesh(
        core_axis_name="c", subcore_axis_name="s", num_cores=NC, num_subcores=NS)

    @functools.partial(
        pl.kernel,
        out_type=jax.ShapeDtypeStruct((B,), jnp.float32),
        mesh=mesh,
        compiler_params=pltpu.CompilerParams(needs_layout_passes=False),
        scratch_types=[
            pltpu.VMEM((NT, D), jnp.float32),
            pltpu.VMEM((NI, D), jnp.float32),
            pltpu.VMEM((NCN, D), jnp.float32),
            pltpu.VMEM((D,), jnp.float32),
            pltpu.VMEM((5, E), jnp.int32),
            pltpu.VMEM((E, D), jnp.float32),
            pltpu.VMEM((E, D), jnp.float32),
            pltpu.VMEM((E,), jnp.float32),
            pltpu.SemaphoreType.DMA,
            pltpu.SemaphoreType.DMA,
        ],
    )
    def sc_score(u_hbm, v_hbm, tp_hbm, ip_hbm, cp_hbm, w2_hbm, idx_hbm, out_hbm,
                 tp_v, ip_v, cp_v, w2_v, idx_v, u_v, v_v, out_v, sem_u, sem_v):
        wid = lax.axis_index("s") * NC + lax.axis_index("c")
        base = wid * per_w
        pltpu.sync_copy(tp_hbm, tp_v)
        pltpu.sync_copy(ip_hbm, ip_v)
        pltpu.sync_copy(cp_hbm, cp_v)
        pltpu.sync_copy(w2_hbm, w2_v)

        def chunk_body(k, carry):
            cbase = base + k * E
            pltpu.sync_copy(idx_hbm.at[wid * chunks + k], idx_v)
            cu = pltpu.async_copy(u_hbm.at[idx_v.at[0]], u_v, sem_u)
            cv = pltpu.async_copy(v_hbm.at[idx_v.at[1]], v_v, sem_v)
            cu.wait()
            cv.wait()
            for g in range(groups):
                rows = lax.iota(jnp.int32, L) + (g * L)
                tvec = idx_v[2, pl.ds(g * L, L)]
                ivec = idx_v[3, pl.ds(g * L, L)]
                cvec = idx_v[4, pl.ds(g * L, L)]

                def jbody(j, acc):
                    jv = jnp.full((L,), 0, jnp.int32) + j
                    u = plsc.load_gather(u_v, [rows, jv])
                    v = plsc.load_gather(v_v, [rows, jv])
                    t = plsc.load_gather(tp_v, [tvec, jv])
                    iv = plsc.load_gather(ip_v, [ivec, jv])
                    c = plsc.load_gather(cp_v, [cvec, jv])
                    w2 = plsc.load_gather(w2_v, [jv])
                    z = u + v + t + iv + c
                    return acc + jnp.maximum(z, 0.0) * w2

                acc = lax.fori_loop(0, D, jbody, jnp.zeros((L,), jnp.float32))
                out_v[pl.ds(g * L, L)] = acc
            pltpu.sync_copy(out_v, out_hbm.at[pl.ds(cbase, E)])
            return carry

        lax.fori_loop(0, chunks, chunk_body, 0)

    return sc_score


def kernel(hidden, times_table, interval_table, connection_table, W1, b1, W2,
           b2, source, destination, times, intervals, connection_types):
    N, D = hidden.shape
    TH = times_table.shape[1]
    IH = interval_table.shape[1]
    B = source.shape[0]
    NT = times_table.shape[0]
    NI = interval_table.shape[0]
    NCN = connection_table.shape[0]
    E = 80
    assert B % (NW * E) == 0

    W1s = W1[:D]
    W1d = W1[D:2 * D]
    W1t = W1[2 * D:2 * D + TH]
    W1i = W1[2 * D + TH:2 * D + TH + IH]
    W1c = W1[2 * D + TH + IH:]

    U, V = _premul_nodes(hidden, W1s, W1d)
    Tp, Ip, Cp = _premul_side(times_table, interval_table, connection_table,
                              W1t, W1i, W1c, b1)

    idx = jnp.stack([source, destination, times, intervals, connection_types])
    idx_blocks = idx.reshape(5, B // E, E).transpose(1, 0, 2)

    sc_score = _make_sc_score(B, D, NT, NI, NCN, E)
    scores = sc_score(U, V, Tp, Ip, Cp, W2.reshape(D), idx_blocks)
    return scores + b2[0]


# speedup vs baseline: 1.4399x; 1.0782x over previous
import functools

import jax
import jax.numpy as jnp
from jax import lax
from jax.experimental import pallas as pl
from jax.experimental.pallas import tpu as pltpu
from jax.experimental.pallas import tpu_sc as plsc

NC = 2
NS = 16
NW = NC * NS
L = 16


def _premul_nodes(hidden, W1s, W1d):
    N, D = hidden.shape
    R = 1000
    assert N % R == 0

    def body(h_ref, ws_ref, wd_ref, u_ref, v_ref):
        h = h_ref[...]
        u_ref[...] = jnp.dot(
            h, ws_ref[...], preferred_element_type=jnp.float32
        ).astype(jnp.bfloat16)
        v_ref[...] = jnp.dot(
            h, wd_ref[...], preferred_element_type=jnp.float32
        ).astype(jnp.bfloat16)

    return pl.pallas_call(
        body,
        grid=(N // R,),
        in_specs=[
            pl.BlockSpec((R, D), lambda i: (i, 0)),
            pl.BlockSpec((D, D), lambda i: (0, 0)),
            pl.BlockSpec((D, D), lambda i: (0, 0)),
        ],
        out_specs=[pl.BlockSpec((R, D), lambda i: (i, 0))] * 2,
        out_shape=[jax.ShapeDtypeStruct((N, D), jnp.bfloat16)] * 2,
    )(hidden, W1s, W1d)


def _premul_side(times_table, interval_table, connection_table, W1t, W1i, W1c, b1):
    NT, TH = times_table.shape
    NI = interval_table.shape[0]
    NCN = connection_table.shape[0]
    D = W1t.shape[1]

    def body(tt, it, ct, wt, wi, wc, b1r, tp, ip, cp):
        b = b1r[...]
        tp[...] = (jnp.dot(tt[...], wt[...], preferred_element_type=jnp.float32)
                   + b).astype(jnp.bfloat16)
        ip[...] = jnp.dot(
            it[...], wi[...], preferred_element_type=jnp.float32
        ).astype(jnp.bfloat16)
        cp[...] = jnp.dot(
            ct[...], wc[...], preferred_element_type=jnp.float32
        ).astype(jnp.bfloat16)

    return pl.pallas_call(
        body,
        out_shape=[
            jax.ShapeDtypeStruct((NT, D), jnp.bfloat16),
            jax.ShapeDtypeStruct((NI, D), jnp.bfloat16),
            jax.ShapeDtypeStruct((NCN, D), jnp.bfloat16),
        ],
    )(times_table, interval_table, connection_table, W1t, W1i, W1c,
      b1.reshape(1, D))


def _pack_pairs(x):
    return lax.bitcast_convert_type(
        x.reshape(x.shape[0], x.shape[1] // 2, 2), jnp.int32)


def _make_sc_score(B, DP, NT, NI, NCN, E):
    per_w = B // NW
    chunks = per_w // E
    groups = E // L
    mesh = plsc.VectorSubcoreMesh(
        core_axis_name="c", subcore_axis_name="s", num_cores=NC, num_subcores=NS)

    @functools.partial(
        pl.kernel,
        out_type=jax.ShapeDtypeStruct((B,), jnp.float32),
        mesh=mesh,
        compiler_params=pltpu.CompilerParams(
            needs_layout_passes=False, use_tc_tiling_on_sc=False),
        scratch_types=[
            pltpu.VMEM((NT, DP), jnp.int32),
            pltpu.VMEM((NI, DP), jnp.int32),
            pltpu.VMEM((NCN, DP), jnp.int32),
            pltpu.VMEM((DP,), jnp.int32),
            pltpu.VMEM((5, E), jnp.int32),
            pltpu.VMEM((5, E), jnp.int32),
            pltpu.VMEM((E, DP), jnp.int32),
            pltpu.VMEM((E, DP), jnp.int32),
            pltpu.VMEM((E, DP), jnp.int32),
            pltpu.VMEM((E, DP), jnp.int32),
            pltpu.VMEM((2 * E,), jnp.float32),
            pltpu.SemaphoreType.DMA,
            pltpu.SemaphoreType.DMA,
            pltpu.SemaphoreType.DMA,
            pltpu.SemaphoreType.DMA,
        ],
    )
    def sc_score(u_hbm, v_hbm, tp_hbm, ip_hbm, cp_hbm, w2_hbm, idx_hbm, out_hbm,
                 tp_v, ip_v, cp_v, w2_v, idx0, idx1, u0, u1, v0, v1, out_v,
                 semu0, semu1, semv0, semv1):
        wid = lax.axis_index("s") * NC + lax.axis_index("c")
        base = wid * per_w
        cb0 = wid * chunks
        pltpu.sync_copy(tp_hbm, tp_v)
        pltpu.sync_copy(ip_hbm, ip_v)
        pltpu.sync_copy(cp_hbm, cp_v)
        pltpu.sync_copy(w2_hbm, w2_v)

        def issue(k, idx_v, u_v, v_v, semu, semv):
            pltpu.sync_copy(idx_hbm.at[cb0 + k], idx_v)
            pltpu.async_copy(u_hbm.at[idx_v.at[0]], u_v, semu)
            pltpu.async_copy(v_hbm.at[idx_v.at[1]], v_v, semv)

        def drain(u_v, v_v, semu, semv):
            pltpu.make_async_copy(u_hbm.at[pl.ds(0, E)], u_v, semu).wait()
            pltpu.make_async_copy(v_hbm.at[pl.ds(0, E)], v_v, semv).wait()

        def compute(idx_v, u_v, v_v, obase):
            for g in range(groups):
                rows = lax.iota(jnp.int32, L) + (g * L)
                tvec = idx_v[2, pl.ds(g * L, L)]
                ivec = idx_v[3, pl.ds(g * L, L)]
                cvec = idx_v[4, pl.ds(g * L, L)]

                def jbody(jp, acc):
                    jv = jnp.full((L,), 0, jnp.int32) + jp
                    ub = plsc.bitcast(plsc.load_gather(u_v, [rows, jv]),
                                      jnp.bfloat16)
                    vb = plsc.bitcast(plsc.load_gather(v_v, [rows, jv]),
                                      jnp.bfloat16)
                    tb = plsc.bitcast(plsc.load_gather(tp_v, [tvec, jv]),
                                      jnp.bfloat16)
                    ib = plsc.bitcast(plsc.load_gather(ip_v, [ivec, jv]),
                                      jnp.bfloat16)
                    cb = plsc.bitcast(plsc.load_gather(cp_v, [cvec, jv]),
                                      jnp.bfloat16)
                    w2b = plsc.bitcast(plsc.load_gather(w2_v, [jv]),
                                       jnp.bfloat16)
                    z = ((ub + vb) + (tb + ib)) + cb
                    p = jnp.maximum(z, jnp.bfloat16(0)) * w2b
                    pa, pb = plsc.unpack(p, format=plsc.PackFormat.INTERLEAVED,
                                         preferred_element_type=jnp.float32)
                    return acc + (pa + pb)

                acc = lax.fori_loop(0, DP, jbody, jnp.zeros((L,), jnp.float32),
                                    unroll=8)
                out_v[pl.ds(obase + g * L, L)] = acc

        issue(0, idx0, u0, v0, semu0, semv0)

        def pair_body(m, carry):
            a = 2 * m
            issue(a + 1, idx1, u1, v1, semu1, semv1)
            drain(u0, v0, semu0, semv0)
            compute(idx0, u0, v0, 0)
            issue(a + 2, idx0, u0, v0, semu0, semv0)
            drain(u1, v1, semu1, semv1)
            compute(idx1, u1, v1, E)
            pltpu.sync_copy(out_v, out_hbm.at[pl.ds(base + a * E, 2 * E)])
            return carry

        lax.fori_loop(0, (chunks - 1) // 2, pair_body, 0)
        drain(u0, v0, semu0, semv0)
        compute(idx0, u0, v0, 0)
        pltpu.sync_copy(out_v.at[pl.ds(0, E)],
                        out_hbm.at[pl.ds(base + (chunks - 1) * E, E)])

    return sc_score


def kernel(hidden, times_table, interval_table, connection_table, W1, b1, W2,
           b2, source, destination, times, intervals, connection_types):
    N, D = hidden.shape
    TH = times_table.shape[1]
    IH = interval_table.shape[1]
    B = source.shape[0]
    NT = times_table.shape[0]
    NI = interval_table.shape[0]
    NCN = connection_table.shape[0]
    E = 80
    assert B % (NW * E) == 0 and (B // (NW * E)) % 2 == 1

    W1s = W1[:D]
    W1d = W1[D:2 * D]
    W1t = W1[2 * D:2 * D + TH]
    W1i = W1[2 * D + TH:2 * D + TH + IH]
    W1c = W1[2 * D + TH + IH:]

    U, V = _premul_nodes(hidden, W1s, W1d)
    Tp, Ip, Cp = _premul_side(times_table, interval_table, connection_table,
                              W1t, W1i, W1c, b1)
    Up, Vp = _pack_pairs(U), _pack_pairs(V)
    Tpp, Ipp, Cpp = _pack_pairs(Tp), _pack_pairs(Ip), _pack_pairs(Cp)
    W2p = _pack_pairs(W2.reshape(1, D).astype(jnp.bfloat16)).reshape(D // 2)

    idx = jnp.stack([source, destination, times, intervals, connection_types])
    idx_blocks = idx.reshape(5, B // E, E).transpose(1, 0, 2)

    sc_score = _make_sc_score(B, D // 2, NT, NI, NCN, E)
    scores = sc_score(Up, Vp, Tpp, Ipp, Cpp, W2p, idx_blocks)
    return scores + b2[0]


# speedup vs baseline: 1.6223x; 1.1267x over previous
import functools

import jax
import jax.numpy as jnp
from jax import lax
from jax.experimental import pallas as pl
from jax.experimental.pallas import tpu as pltpu
from jax.experimental.pallas import tpu_sc as plsc

NC = 2
NS = 16
NW = NC * NS
L = 16


def _premul_nodes(hidden, W1s, W1d):
    N, D = hidden.shape
    R = 1000
    assert N % R == 0

    def body(h_ref, ws_ref, wd_ref, u_ref, v_ref):
        h = h_ref[...]
        u_ref[...] = jnp.dot(
            h, ws_ref[...], preferred_element_type=jnp.float32
        ).astype(jnp.bfloat16)
        v_ref[...] = jnp.dot(
            h, wd_ref[...], preferred_element_type=jnp.float32
        ).astype(jnp.bfloat16)

    return pl.pallas_call(
        body,
        grid=(N // R,),
        in_specs=[
            pl.BlockSpec((R, D), lambda i: (i, 0)),
            pl.BlockSpec((D, D), lambda i: (0, 0)),
            pl.BlockSpec((D, D), lambda i: (0, 0)),
        ],
        out_specs=[pl.BlockSpec((R, D), lambda i: (i, 0))] * 2,
        out_shape=[jax.ShapeDtypeStruct((N, D), jnp.bfloat16)] * 2,
    )(hidden, W1s, W1d)


def _premul_side(times_table, interval_table, connection_table, W1t, W1i, W1c, b1):
    NT, TH = times_table.shape
    NI = interval_table.shape[0]
    NCN = connection_table.shape[0]
    D = W1t.shape[1]

    def body(tt, it, ct, wt, wi, wc, b1r, tp, ip, cp):
        b = b1r[...]
        tp[...] = (jnp.dot(tt[...], wt[...], preferred_element_type=jnp.float32)
                   + b).astype(jnp.bfloat16)
        ip[...] = jnp.dot(
            it[...], wi[...], preferred_element_type=jnp.float32
        ).astype(jnp.bfloat16)
        cp[...] = jnp.dot(
            ct[...], wc[...], preferred_element_type=jnp.float32
        ).astype(jnp.bfloat16)

    return pl.pallas_call(
        body,
        out_shape=[
            jax.ShapeDtypeStruct((NT, D), jnp.bfloat16),
            jax.ShapeDtypeStruct((NI, D), jnp.bfloat16),
            jax.ShapeDtypeStruct((NCN, D), jnp.bfloat16),
        ],
    )(times_table, interval_table, connection_table, W1t, W1i, W1c,
      b1.reshape(1, D))


def _pack_pairs(x):
    return lax.bitcast_convert_type(
        x.reshape(x.shape[0], x.shape[1] // 2, 2), jnp.int32)


def _make_sc_gather_sum(B, DP, K, E):
    per_w = B // NW
    chunks = per_w // E
    assert chunks % 2 == 1 and E % L == 0
    mesh = plsc.VectorSubcoreMesh(
        core_axis_name="c", subcore_axis_name="s", num_cores=NC, num_subcores=NS)

    @functools.partial(
        pl.kernel,
        out_type=jax.ShapeDtypeStruct((B, DP), jnp.int32),
        mesh=mesh,
        compiler_params=pltpu.CompilerParams(
            needs_layout_passes=False, use_tc_tiling_on_sc=False),
        scratch_types=[
            pltpu.VMEM((K, DP), jnp.int32),
            pltpu.VMEM((5, E), jnp.int32),
            pltpu.VMEM((5, E), jnp.int32),
            pltpu.VMEM((E, DP), jnp.int32),
            pltpu.VMEM((E, DP), jnp.int32),
            pltpu.VMEM((E, DP), jnp.int32),
            pltpu.VMEM((E, DP), jnp.int32),
            pltpu.VMEM((E, DP), jnp.int32),
            pltpu.VMEM((E, DP), jnp.int32),
            pltpu.SemaphoreType.DMA,
            pltpu.SemaphoreType.DMA,
            pltpu.SemaphoreType.DMA,
            pltpu.SemaphoreType.DMA,
        ],
    )
    def sc_gather_sum(u_hbm, v_hbm, side_hbm, idx_hbm, w_hbm,
                      side_v, idx0, idx1, u0, u1, v0, v1, w0, w1,
                      semg0, semg1, semo0, semo1):
        wid = lax.axis_index("s") * NC + lax.axis_index("c")
        base = wid * per_w
        cb0 = wid * chunks
        pltpu.sync_copy(side_hbm, side_v)

        def issue(k, idx_p, u_p, v_p, semg):
            pltpu.sync_copy(idx_hbm.at[cb0 + k], idx_p)
            pltpu.async_copy(u_hbm.at[idx_p.at[0]], u_p, semg)
            pltpu.async_copy(v_hbm.at[idx_p.at[1]], v_p, semg)

        def drain_g(u_p, v_p, semg):
            pltpu.make_async_copy(u_hbm.at[pl.ds(0, E)], u_p, semg).wait()
            pltpu.make_async_copy(v_hbm.at[pl.ds(0, E)], v_p, semg).wait()

        def drain_o(w_p, semo):
            pltpu.make_async_copy(u_hbm.at[pl.ds(0, E)], w_p, semo).wait()

        def add_rows(idx_p, u_p, v_p, w_p):
            def gbody(g, carry):
                tvec = idx_p[2, pl.ds(g * L, L)]
                ivec = idx_p[3, pl.ds(g * L, L)]
                cvec = idx_p[4, pl.ds(g * L, L)]
                for j in range(L):
                    e = g * L + j
                    t = tvec[j]
                    i = ivec[j]
                    c = cvec[j]
                    for kk in range(DP // L):
                        s = pl.ds(kk * L, L)
                        a = (plsc.bitcast(u_p[e, s], jnp.bfloat16)
                             + plsc.bitcast(v_p[e, s], jnp.bfloat16))
                        b = (plsc.bitcast(side_v[t, s], jnp.bfloat16)
                             + plsc.bitcast(side_v[i, s], jnp.bfloat16))
                        z = (a + b) + plsc.bitcast(side_v[c, s], jnp.bfloat16)
                        w_p[e, s] = plsc.bitcast(z, jnp.int32)
                return carry
            lax.fori_loop(0, E // L, gbody, 0)

        issue(0, idx0, u0, v0, semg0)

        def pair_body(m, carry):
            a = 2 * m
            issue(a + 1, idx1, u1, v1, semg1)

            @pl.when(m > 0)
            def _():
                drain_o(w0, semo0)

            drain_g(u0, v0, semg0)
            add_rows(idx0, u0, v0, w0)
            pltpu.async_copy(w0, w_hbm.at[pl.ds(base + a * E, E)], semo0)
            issue(a + 2, idx0, u0, v0, semg0)

            @pl.when(m > 0)
            def _():
                drain_o(w1, semo1)

            drain_g(u1, v1, semg1)
            add_rows(idx1, u1, v1, w1)
            pltpu.async_copy(w1, w_hbm.at[pl.ds(base + (a + 1) * E, E)], semo1)
            return carry

        lax.fori_loop(0, (chunks - 1) // 2, pair_body, 0)
        drain_o(w0, semo0)
        drain_g(u0, v0, semg0)
        add_rows(idx0, u0, v0, w0)
        pltpu.async_copy(w0, w_hbm.at[pl.ds(base + (chunks - 1) * E, E)], semo0)
        drain_o(w0, semo0)
        drain_o(w1, semo1)

    return sc_gather_sum


def _score_tc(w128, W2):
    B, D = w128.shape
    R = 8000
    assert B % R == 0

    def body(w_ref, w2_ref, o_ref):
        h = jnp.maximum(w_ref[...].astype(jnp.float32), 0.0)
        o_ref[...] = jnp.dot(h, w2_ref[...], preferred_element_type=jnp.float32)

    return pl.pallas_call(
        body,
        grid=(B // R,),
        in_specs=[
            pl.BlockSpec((R, D), lambda k: (k, 0)),
            pl.BlockSpec((D, 1), lambda k: (0, 0)),
        ],
        out_specs=pl.BlockSpec((R, 1), lambda k: (k, 0)),
        out_shape=jax.ShapeDtypeStruct((B, 1), jnp.float32),
    )(w128, W2)


def kernel(hidden, times_table, interval_table, connection_table, W1, b1, W2,
           b2, source, destination, times, intervals, connection_types):
    N, D = hidden.shape
    TH = times_table.shape[1]
    IH = interval_table.shape[1]
    B = source.shape[0]
    NT = times_table.shape[0]
    NI = interval_table.shape[0]
    NCN = connection_table.shape[0]
    E = 80
    assert B % (NW * E) == 0

    W1s = W1[:D]
    W1d = W1[D:2 * D]
    W1t = W1[2 * D:2 * D + TH]
    W1i = W1[2 * D + TH:2 * D + TH + IH]
    W1c = W1[2 * D + TH + IH:]

    U, V = _premul_nodes(hidden, W1s, W1d)
    Tp, Ip, Cp = _premul_side(times_table, interval_table, connection_table,
                              W1t, W1i, W1c, b1)
    Up, Vp = _pack_pairs(U), _pack_pairs(V)

    pad = (-(NT + NI + NCN)) % 8
    sidetab = jnp.concatenate(
        [Tp, Ip, Cp, jnp.zeros((pad, D), jnp.bfloat16)], axis=0)
    sidepk = _pack_pairs(sidetab)
    K = sidetab.shape[0]

    idx = jnp.stack([source, destination, times, intervals + NT,
                     connection_types + (NT + NI)])
    idx_blocks = idx.reshape(5, B // E, E).transpose(1, 0, 2)

    sc_gather_sum = _make_sc_gather_sum(B, D // 2, K, E)
    w_packed = sc_gather_sum(Up, Vp, sidepk, idx_blocks)
    w128 = lax.bitcast_convert_type(w_packed, jnp.bfloat16).reshape(B, D)

    scores = _score_tc(w128, W2)
    return scores[:, 0] + b2[0]


# speedup vs baseline: 6.6472x; 4.0974x over previous
import functools

import jax
import jax.numpy as jnp
from jax import lax
from jax.experimental import pallas as pl
from jax.experimental.pallas import tpu as pltpu
from jax.experimental.pallas import tpu_sc as plsc

NC = 2
NS = 16
NW = NC * NS
L = 16


def _premul_nodes(hidden, W1s, W1d):
    N, D = hidden.shape
    R = 1000
    assert N % R == 0

    def body(h_ref, ws_ref, wd_ref, u_ref, v_ref):
        h = h_ref[...]
        u_ref[...] = jnp.dot(
            h, ws_ref[...], preferred_element_type=jnp.float32
        ).astype(jnp.bfloat16)
        v_ref[...] = jnp.dot(
            h, wd_ref[...], preferred_element_type=jnp.float32
        ).astype(jnp.bfloat16)

    return pl.pallas_call(
        body,
        grid=(N // R,),
        in_specs=[
            pl.BlockSpec((R, D), lambda i: (i, 0)),
            pl.BlockSpec((D, D), lambda i: (0, 0)),
            pl.BlockSpec((D, D), lambda i: (0, 0)),
        ],
        out_specs=[pl.BlockSpec((R, D), lambda i: (i, 0))] * 2,
        out_shape=[jax.ShapeDtypeStruct((N, D), jnp.bfloat16)] * 2,
    )(hidden, W1s, W1d)


def _premul_side(times_table, interval_table, connection_table, W1t, W1i, W1c, b1):
    NT, TH = times_table.shape
    NI = interval_table.shape[0]
    NCN = connection_table.shape[0]
    D = W1t.shape[1]

    def body(tt, it, ct, wt, wi, wc, b1r, tp, ip, cp):
        b = b1r[...]
        tp[...] = (jnp.dot(tt[...], wt[...], preferred_element_type=jnp.float32)
                   + b).astype(jnp.bfloat16)
        ip[...] = jnp.dot(
            it[...], wi[...], preferred_element_type=jnp.float32
        ).astype(jnp.bfloat16)
        cp[...] = jnp.dot(
            ct[...], wc[...], preferred_element_type=jnp.float32
        ).astype(jnp.bfloat16)

    return pl.pallas_call(
        body,
        out_shape=[
            jax.ShapeDtypeStruct((NT, D), jnp.bfloat16),
            jax.ShapeDtypeStruct((NI, D), jnp.bfloat16),
            jax.ShapeDtypeStruct((NCN, D), jnp.bfloat16),
        ],
    )(times_table, interval_table, connection_table, W1t, W1i, W1c,
      b1.reshape(1, D))


def _make_sc_score(B, D, K, E, NT, NI):
    per_w = B // NW
    chunks = per_w // E
    assert chunks % 2 == 1 and E % L == 0
    W = 2 * L
    mesh = plsc.VectorSubcoreMesh(
        core_axis_name="c", subcore_axis_name="s", num_cores=NC, num_subcores=NS)

    @functools.partial(
        pl.kernel,
        out_type=jax.ShapeDtypeStruct((B,), jnp.float32),
        mesh=mesh,
        compiler_params=pltpu.CompilerParams(
            needs_layout_passes=False, use_tc_tiling_on_sc=False),
        scratch_types=[
            pltpu.VMEM((K, D), jnp.bfloat16),
            pltpu.VMEM((D,), jnp.bfloat16),
            pltpu.VMEM((L, L), jnp.float32),
            pltpu.VMEM((5, E), jnp.int32),
            pltpu.VMEM((5, E), jnp.int32),
            pltpu.VMEM((E, D), jnp.bfloat16),
            pltpu.VMEM((E, D), jnp.bfloat16),
            pltpu.VMEM((E, D), jnp.bfloat16),
            pltpu.VMEM((E, D), jnp.bfloat16),
            pltpu.VMEM((E,), jnp.float32),
            pltpu.VMEM((E,), jnp.float32),
            pltpu.SemaphoreType.DMA,
            pltpu.SemaphoreType.DMA,
            pltpu.SemaphoreType.DMA,
            pltpu.SemaphoreType.DMA,
            pltpu.SemaphoreType.DMA,
            pltpu.SemaphoreType.DMA,
        ],
    )
    def sc_score(u_hbm, v_hbm, side_hbm, w2_hbm,
                 src_hbm, dst_hbm, t_hbm, i_hbm, c_hbm, s_hbm,
                 side_v, w2_v, mat_v, idx0, idx1, u0, u1, v0, v1, s0, s1,
                 semi0, semi1, semg0, semg1, semo0, semo1):
        wid = lax.axis_index("s") * NC + lax.axis_index("c")
        base = wid * per_w
        pltpu.sync_copy(side_hbm, side_v)
        pltpu.sync_copy(w2_hbm, w2_v)
        w2b = [w2_v[pl.ds(kk * W, W)] for kk in range(D // W)]
        lanes = lax.iota(jnp.int32, L)
        streams = (src_hbm, dst_hbm, t_hbm, i_hbm, c_hbm)

        def issue(k, idx_p, u_p, v_p, semi, semg):
            cbase = base + k * E
            for r in range(5):
                pltpu.async_copy(
                    streams[r].at[pl.ds(cbase, E)], idx_p.at[r], semi)
            for r in range(5):
                pltpu.make_async_copy(
                    streams[r].at[pl.ds(0, E)], idx_p.at[r], semi).wait()
            pltpu.async_copy(u_hbm.at[idx_p.at[0]], u_p, semg)
            pltpu.async_copy(v_hbm.at[idx_p.at[1]], v_p, semg)

        def drain_g(u_p, v_p, semg):
            pltpu.make_async_copy(u_hbm.at[pl.ds(0, E)], u_p, semg).wait()
            pltpu.make_async_copy(v_hbm.at[pl.ds(0, E)], v_p, semg).wait()

        def drain_o(s_p, semo):
            pltpu.make_async_copy(s_hbm.at[pl.ds(0, E)], s_p, semo).wait()

        def score_chunk(idx_p, u_p, v_p, s_p):
            def gbody(g, carry):
                tvec = idx_p[2, pl.ds(g * L, L)]
                ivec = idx_p[3, pl.ds(g * L, L)] + NT
                cvec = idx_p[4, pl.ds(g * L, L)] + (NT + NI)
                for j in range(L):
                    e = g * L + j
                    t = tvec[j]
                    i = ivec[j]
                    c = cvec[j]
                    parts = []
                    for kk in range(D // W):
                        s = pl.ds(kk * W, W)
                        a = u_p[e, s] + v_p[e, s]
                        b = side_v[t, s] + side_v[i, s]
                        z = (a + b) + side_v[c, s]
                        p = jnp.maximum(z, jnp.bfloat16(0)) * w2b[kk]
                        pa, pb = plsc.unpack(
                            p, format=plsc.PackFormat.INTERLEAVED,
                            preferred_element_type=jnp.float32)
                        parts.append(pa + pb)
                    mat_v[j, :] = (parts[0] + parts[1]) + (parts[2] + parts[3])
                acc = plsc.load_gather(
                    mat_v, [lanes, jnp.full((L,), 0, jnp.int32)])
                for col in range(1, L):
                    acc = acc + plsc.load_gather(
                        mat_v, [lanes, jnp.full((L,), col, jnp.int32)])
                s_p[pl.ds(g * L, L)] = acc
                return carry
            lax.fori_loop(0, E // L, gbody, 0)

        issue(0, idx0, u0, v0, semi0, semg0)

        def pair_body(m, carry):
            a = 2 * m
            issue(a + 1, idx1, u1, v1, semi1, semg1)

            @pl.when(m > 0)
            def _():
                drain_o(s0, semo0)

            drain_g(u0, v0, semg0)
            score_chunk(idx0, u0, v0, s0)
            pltpu.async_copy(s0, s_hbm.at[pl.ds(base + a * E, E)], semo0)
            issue(a + 2, idx0, u0, v0, semi0, semg0)

            @pl.when(m > 0)
            def _():
                drain_o(s1, semo1)

            drain_g(u1, v1, semg1)
            score_chunk(idx1, u1, v1, s1)
            pltpu.async_copy(s1, s_hbm.at[pl.ds(base + (a + 1) * E, E)], semo1)
            return carry

        lax.fori_loop(0, (chunks - 1) // 2, pair_body, 0)
        drain_o(s0, semo0)
        drain_g(u0, v0, semg0)
        score_chunk(idx0, u0, v0, s0)
        pltpu.async_copy(s0, s_hbm.at[pl.ds(base + (chunks - 1) * E, E)], semo0)
        drain_o(s0, semo0)
        drain_o(s1, semo1)

    return sc_score


def kernel(hidden, times_table, interval_table, connection_table, W1, b1, W2,
           b2, source, destination, times, intervals, connection_types):
    N, D = hidden.shape
    TH = times_table.shape[1]
    IH = interval_table.shape[1]
    B = source.shape[0]
    NT = times_table.shape[0]
    NI = interval_table.shape[0]
    NCN = connection_table.shape[0]
    E = 80
    assert B % (NW * E) == 0

    W1s = W1[:D]
    W1d = W1[D:2 * D]
    W1t = W1[2 * D:2 * D + TH]
    W1i = W1[2 * D + TH:2 * D + TH + IH]
    W1c = W1[2 * D + TH + IH:]

    U, V = _premul_nodes(hidden, W1s, W1d)
    Tp, Ip, Cp = _premul_side(times_table, interval_table, connection_table,
                              W1t, W1i, W1c, b1)

    pad = (-(NT + NI + NCN)) % 8
    sidetab = jnp.concatenate(
        [Tp, Ip, Cp, jnp.zeros((pad, D), jnp.bfloat16)], axis=0)
    K = sidetab.shape[0]

    sc_score = _make_sc_score(B, D, K, E, NT, NI)
    scores = sc_score(U, V, sidetab, W2.reshape(D).astype(jnp.bfloat16),
                      source, destination, times, intervals, connection_types)
    return scores + b2[0]


# speedup vs baseline: 7.0194x; 1.0560x over previous
import functools

import jax
import jax.numpy as jnp
from jax import lax
from jax.experimental import pallas as pl
from jax.experimental.pallas import tpu as pltpu
from jax.experimental.pallas import tpu_sc as plsc

NC = 2
NS = 16
NW = NC * NS
L = 16


def _premul_nodes(hidden, W1):
    N, D = hidden.shape
    R = 2000
    assert N % R == 0

    def body(h_ref, w1_ref, u_ref, v_ref):
        h = h_ref[...]
        w1 = w1_ref[...]
        u_ref[...] = jnp.dot(
            h, w1[0:D], preferred_element_type=jnp.float32
        ).astype(jnp.bfloat16)
        v_ref[...] = jnp.dot(
            h, w1[D:2 * D], preferred_element_type=jnp.float32
        ).astype(jnp.bfloat16)

    W1K = W1.shape[0]
    return pl.pallas_call(
        body,
        grid=(N // R,),
        in_specs=[
            pl.BlockSpec((R, D), lambda i: (i, 0)),
            pl.BlockSpec((W1K, D), lambda i: (0, 0)),
        ],
        out_specs=[pl.BlockSpec((R, D), lambda i: (i, 0))] * 2,
        out_shape=[jax.ShapeDtypeStruct((N, D), jnp.bfloat16)] * 2,
    )(hidden, W1)


def _premul_side(times_table, interval_table, connection_table, W1, b1, K):
    NT, TH = times_table.shape
    NI, IH = interval_table.shape
    NCN = connection_table.shape[0]
    D = W1.shape[1]

    def body(tt, it, ct, w1_ref, b1r, tab):
        w1 = w1_ref[...]
        wt = w1[2 * D:2 * D + TH]
        wi = w1[2 * D + TH:2 * D + TH + IH]
        wc = w1[2 * D + TH + IH:]
        tab[0:NT] = (jnp.dot(tt[...], wt, preferred_element_type=jnp.float32)
                     + b1r[...]).astype(jnp.bfloat16)
        tab[NT:NT + NI] = jnp.dot(
            it[...], wi, preferred_element_type=jnp.float32
        ).astype(jnp.bfloat16)
        tab[NT + NI:NT + NI + NCN] = jnp.dot(
            ct[...], wc, preferred_element_type=jnp.float32
        ).astype(jnp.bfloat16)
        tab[NT + NI + NCN:K] = jnp.zeros(
            (K - NT - NI - NCN, D), jnp.bfloat16)

    return pl.pallas_call(
        body,
        out_shape=jax.ShapeDtypeStruct((K, D), jnp.bfloat16),
    )(times_table, interval_table, connection_table, W1, b1.reshape(1, D))


def _make_sc_score(B, D, K, E, NT, NI):
    per_w = B // NW
    chunks = per_w // E
    assert chunks % 2 == 1 and E % L == 0
    W = 2 * L
    mesh = plsc.VectorSubcoreMesh(
        core_axis_name="c", subcore_axis_name="s", num_cores=NC, num_subcores=NS)

    @functools.partial(
        pl.kernel,
        out_type=jax.ShapeDtypeStruct((B,), jnp.float32),
        mesh=mesh,
        compiler_params=pltpu.CompilerParams(
            needs_layout_passes=False, use_tc_tiling_on_sc=False),
        scratch_types=[
            pltpu.VMEM((K, D), jnp.bfloat16),
            pltpu.VMEM((D,), jnp.bfloat16),
            pltpu.VMEM((L,), jnp.float32),
            pltpu.VMEM((L, L), jnp.float32),
            pltpu.VMEM((5, E), jnp.int32),
            pltpu.VMEM((5, E), jnp.int32),
            pltpu.VMEM((E, D), jnp.bfloat16),
            pltpu.VMEM((E, D), jnp.bfloat16),
            pltpu.VMEM((E, D), jnp.bfloat16),
            pltpu.VMEM((E, D), jnp.bfloat16),
            pltpu.VMEM((E,), jnp.float32),
            pltpu.VMEM((E,), jnp.float32),
            pltpu.SemaphoreType.DMA,
            pltpu.SemaphoreType.DMA,
            pltpu.SemaphoreType.DMA,
            pltpu.SemaphoreType.DMA,
            pltpu.SemaphoreType.DMA,
            pltpu.SemaphoreType.DMA,
        ],
    )
    def sc_score(u_hbm, v_hbm, side_hbm, w2_hbm, b2_hbm,
                 src_hbm, dst_hbm, t_hbm, i_hbm, c_hbm, s_hbm,
                 side_v, w2_v, b2_v, mat_v, idx0, idx1, u0, u1, v0, v1, s0, s1,
                 semi0, semi1, semg0, semg1, semo0, semo1):
        wid = lax.axis_index("s") * NC + lax.axis_index("c")
        base = wid * per_w
        pltpu.sync_copy(side_hbm, side_v)
        pltpu.sync_copy(w2_hbm, w2_v)
        pltpu.sync_copy(b2_hbm, b2_v)
        w2b = [w2_v[pl.ds(kk * W, W)] for kk in range(D // W)]
        lanes = lax.iota(jnp.int32, L)
        streams = (src_hbm, dst_hbm, t_hbm, i_hbm, c_hbm)

        def issue(k, idx_p, u_p, v_p, semi, semg):
            cbase = base + k * E
            for r in range(5):
                pltpu.async_copy(
                    streams[r].at[pl.ds(cbase, E)], idx_p.at[r], semi)
            for r in range(5):
                pltpu.make_async_copy(
                    streams[r].at[pl.ds(0, E)], idx_p.at[r], semi).wait()
            pltpu.async_copy(u_hbm.at[idx_p.at[0]], u_p, semg)
            pltpu.async_copy(v_hbm.at[idx_p.at[1]], v_p, semg)

        def drain_g(u_p, v_p, semg):
            pltpu.make_async_copy(u_hbm.at[pl.ds(0, E)], u_p, semg).wait()
            pltpu.make_async_copy(v_hbm.at[pl.ds(0, E)], v_p, semg).wait()

        def drain_o(s_p, semo):
            pltpu.make_async_copy(s_hbm.at[pl.ds(0, E)], s_p, semo).wait()

        def score_chunk(idx_p, u_p, v_p, s_p):
            def gbody(g, carry):
                tvec = idx_p[2, pl.ds(g * L, L)]
                ivec = idx_p[3, pl.ds(g * L, L)] + NT
                cvec = idx_p[4, pl.ds(g * L, L)] + (NT + NI)
                for j in range(L):
                    e = g * L + j
                    t = tvec[j]
                    i = ivec[j]
                    c = cvec[j]
                    parts = []
                    for kk in range(D // W):
                        s = pl.ds(kk * W, W)
                        a = u_p[e, s] + v_p[e, s]
                        b = side_v[t, s] + side_v[i, s]
                        z = (a + b) + side_v[c, s]
                        p = jnp.maximum(z, jnp.bfloat16(0)) * w2b[kk]
                        pa, pb = plsc.unpack(
                            p, format=plsc.PackFormat.INTERLEAVED,
                            preferred_element_type=jnp.float32)
                        parts.append(pa + pb)
                    mat_v[j, :] = (parts[0] + parts[1]) + (parts[2] + parts[3])
                acc = b2_v[...] + plsc.load_gather(
                    mat_v, [lanes, jnp.full((L,), 0, jnp.int32)])
                for col in range(1, L):
                    acc = acc + plsc.load_gather(
                        mat_v, [lanes, jnp.full((L,), col, jnp.int32)])
                s_p[pl.ds(g * L, L)] = acc
                return carry
            lax.fori_loop(0, E // L, gbody, 0)

        issue(0, idx0, u0, v0, semi0, semg0)

        def pair_body(m, carry):
            a = 2 * m
            issue(a + 1, idx1, u1, v1, semi1, semg1)

            @pl.when(m > 0)
            def _():
                drain_o(s0, semo0)

            drain_g(u0, v0, semg0)
            score_chunk(idx0, u0, v0, s0)
            pltpu.async_copy(s0, s_hbm.at[pl.ds(base + a * E, E)], semo0)
            issue(a + 2, idx0, u0, v0, semi0, semg0)

            @pl.when(m > 0)
            def _():
                drain_o(s1, semo1)

            drain_g(u1, v1, semg1)
            score_chunk(idx1, u1, v1, s1)
            pltpu.async_copy(s1, s_hbm.at[pl.ds(base + (a + 1) * E, E)], semo1)
            return carry

        lax.fori_loop(0, (chunks - 1) // 2, pair_body, 0)
        drain_o(s0, semo0)
        drain_g(u0, v0, semg0)
        score_chunk(idx0, u0, v0, s0)
        pltpu.async_copy(s0, s_hbm.at[pl.ds(base + (chunks - 1) * E, E)], semo0)
        drain_o(s0, semo0)
        drain_o(s1, semo1)

    return sc_score


def kernel(hidden, times_table, interval_table, connection_table, W1, b1, W2,
           b2, source, destination, times, intervals, connection_types):
    N, D = hidden.shape
    TH = times_table.shape[1]
    IH = interval_table.shape[1]
    B = source.shape[0]
    NT = times_table.shape[0]
    NI = interval_table.shape[0]
    NCN = connection_table.shape[0]
    E = 80
    assert B % (NW * E) == 0

    U, V = _premul_nodes(hidden, W1)
    K = NT + NI + NCN + ((-(NT + NI + NCN)) % 8)
    sidetab = _premul_side(times_table, interval_table, connection_table,
                           W1, b1, K)

    sc_score = _make_sc_score(B, D, K, E, NT, NI)
    scores = sc_score(U, V, sidetab, W2.reshape(D).astype(jnp.bfloat16),
                      jnp.broadcast_to(b2, (L,)),
                      source, destination, times, intervals, connection_types)
    return scores
